# Initial kernel scaffold; baseline (speedup 1.0000x reference)
import jax
import jax.numpy as jnp
from jax.experimental import pallas as pl


def kernel(inputs):
    raise NotImplementedError("write your pallas kernel here")


# speedup vs baseline: 2.0444x; 2.0444x over previous
import functools

import jax
import jax.numpy as jnp
from jax import lax
from jax.experimental import pallas as pl
from jax.experimental.pallas import tpu as pltpu
from jax.experimental.pallas import tpu_sc as plsc

NUM_CORES = 2
NUM_SUBCORES = 16
NUM_WORKERS = NUM_CORES * NUM_SUBCORES
B = 128
N_PER_BATCH = 1024 * 16 * 16
BATCHES_PER_WORKER = B // NUM_WORKERS
CHUNK = 32768
CHUNKS_PER_BATCH = N_PER_BATCH // CHUNK
TOTAL_CHUNKS = BATCHES_PER_WORKER * CHUNKS_PER_BATCH
HIST = 256 * 16


def _make_kernel():
    mesh = plsc.VectorSubcoreMesh(
        core_axis_name="c", subcore_axis_name="s", num_cores=NUM_CORES
    )

    @functools.partial(
        pl.kernel,
        mesh=mesh,
        out_type=jax.ShapeDtypeStruct((B, 16), jnp.float32),
        compiler_params=pltpu.CompilerParams(needs_layout_passes=False),
        scratch_types=[
            pltpu.VMEM((CHUNK,), jnp.int32),
            pltpu.VMEM((CHUNK,), jnp.int32),
            pltpu.VMEM((HIST,), jnp.float32),
            pltpu.VMEM((16,), jnp.float32),
            pltpu.SemaphoreType.DMA,
            pltpu.SemaphoreType.DMA,
        ],
    )
    def hist_kernel(x_hbm, out_hbm, buf0, buf1, hist, res, sem0, sem1):
        w = lax.axis_index("s") * NUM_CORES + lax.axis_index("c")
        lane = lax.iota(jnp.int32, 16)
        ones = jnp.ones((16,), jnp.float32)
        fzero = jnp.zeros((16,), jnp.float32)
        bufs = (buf0, buf1)
        sems = (sem0, sem1)
        base = w * BATCHES_PER_WORKER * N_PER_BATCH

        def start(t):
            return pltpu.async_copy(
                x_hbm.at[pl.ds(base + t * CHUNK, CHUNK)], bufs[t % 2], sems[t % 2]
            )

        def zero_hist():
            @plsc.parallel_loop(0, HIST, 16, unroll=8)
            def _(i):
                hist[pl.ds(i, 16)] = fzero

        zero_hist()
        pending = start(0)
        for j in range(BATCHES_PER_WORKER):
            b = w * BATCHES_PER_WORKER + j
            for c in range(CHUNKS_PER_BATCH):
                t = j * CHUNKS_PER_BATCH + c
                pending.wait()
                if t + 1 < TOTAL_CHUNKS:
                    pending = start(t + 1)
                buf = bufs[t % 2]

                @plsc.parallel_loop(0, CHUNK, 16, unroll=8)
                def _(i):
                    vals = buf[pl.ds(i, 16)]
                    idx = (vals << 4) + lane
                    plsc.addupdate_scatter(hist, [idx], ones)

            def rbody(v, m):
                row = hist[pl.ds(v * 16, 16)]
                return jnp.maximum(m, jnp.sum(row))

            mx = lax.fori_loop(0, 256, rbody, jnp.float32(0.0), unroll=4)
            res[...] = jnp.full((16,), mx, jnp.float32)
            pltpu.sync_copy(res, out_hbm.at[b])
            if j + 1 < BATCHES_PER_WORKER:
                zero_hist()

    return hist_kernel


_hist_kernel = _make_kernel()


def kernel(inputs):
    x = inputs.reshape(-1)
    padded = _hist_kernel(x)
    return padded[:, 0]


# speedup vs baseline: 16.9405x; 8.2862x over previous
import functools

import jax
import jax.numpy as jnp
from jax import lax
from jax.experimental import pallas as pl
from jax.experimental.pallas import tpu as pltpu
from jax.experimental.pallas import tpu_sc as plsc

NUM_CORES = 2
NUM_SUBCORES = 16
NUM_WORKERS = NUM_CORES * NUM_SUBCORES
B = 128
ROW = 1024
ROWS_PER_BATCH = 256
BATCHES_PER_WORKER = B // NUM_WORKERS
CHUNK_ROWS = 32
CHUNK = CHUNK_ROWS * ROW
CHUNKS_PER_BATCH = ROWS_PER_BATCH // CHUNK_ROWS
TOTAL_CHUNKS = BATCHES_PER_WORKER * CHUNKS_PER_BATCH
HIST = 256 * 16


def _make_kernel():
    mesh = plsc.VectorSubcoreMesh(
        core_axis_name="c", subcore_axis_name="s", num_cores=NUM_CORES
    )

    @functools.partial(
        pl.kernel,
        mesh=mesh,
        out_type=jax.ShapeDtypeStruct((B, 16), jnp.float32),
        compiler_params=pltpu.CompilerParams(needs_layout_passes=False),
        scratch_types=[
            pltpu.VMEM((CHUNK_ROWS, ROW), jnp.int32),
            pltpu.VMEM((CHUNK_ROWS, ROW), jnp.int32),
            pltpu.VMEM((HIST,), jnp.float32),
            pltpu.VMEM((16,), jnp.float32),
            pltpu.SemaphoreType.DMA,
            pltpu.SemaphoreType.DMA,
        ],
    )
    def hist_kernel(x_hbm, out_hbm, buf0, buf1, hist, res, sem0, sem1):
        w = lax.axis_index("s") * NUM_CORES + lax.axis_index("c")
        lane = lax.iota(jnp.int32, 16)
        ones = jnp.ones((16,), jnp.float32)
        fzero = jnp.zeros((16,), jnp.float32)
        bufs = (buf0, buf1)
        sems = (sem0, sem1)
        row0 = w * BATCHES_PER_WORKER * ROWS_PER_BATCH

        def start(t):
            return pltpu.async_copy(
                x_hbm.at[pl.ds(row0 + t * CHUNK_ROWS, CHUNK_ROWS)],
                bufs[t % 2],
                sems[t % 2],
            )

        def zero_hist():
            @plsc.parallel_loop(0, HIST, 16, unroll=8)
            def _(i):
                hist[pl.ds(i, 16)] = fzero

        zero_hist()
        pending = start(0)
        for j in range(BATCHES_PER_WORKER):
            b = w * BATCHES_PER_WORKER + j
            for c in range(CHUNKS_PER_BATCH):
                t = j * CHUNKS_PER_BATCH + c
                pending.wait()
                if t + 1 < TOTAL_CHUNKS:
                    pending = start(t + 1)
                buf = bufs[t % 2]

                @plsc.parallel_loop(0, CHUNK // 16, 1, unroll=8)
                def _(i):
                    r = i >> 6
                    col = (i & 63) << 4
                    vals = buf[r, pl.ds(col, 16)]
                    idx = (vals << 4) + lane
                    plsc.addupdate_scatter(hist, [idx], ones)

            def rbody(v, m):
                row = hist[pl.ds(v * 16, 16)]
                return jnp.maximum(m, jnp.sum(row))

            mx = lax.fori_loop(0, 256, rbody, jnp.float32(0.0), unroll=4)
            res[...] = jnp.full((16,), mx, jnp.float32)
            pltpu.sync_copy(res, out_hbm.at[b])
            if j + 1 < BATCHES_PER_WORKER:
                zero_hist()

    return hist_kernel


_hist_kernel = _make_kernel()


def kernel(inputs):
    x = inputs.transpose(0, 2, 3, 1).reshape(B * ROWS_PER_BATCH, ROW)
    padded = _hist_kernel(x)
    return padded[:, 0]
